# Initial kernel scaffold; baseline (speedup 1.0000x reference)
import functools
import math

import jax
import jax.numpy as jnp
from jax import lax
from jax.experimental import pallas as pl
from jax.experimental.pallas import tpu as pltpu
from jax.experimental.pallas import tpu_sc as plsc

_NC = 2
_NS = 16
_NW = _NC * _NS
_LANES = 16


def _make_comb(pe, seg_table):
    Ls, D = pe.shape
    S = seg_table.shape[0]

    def body(pe_ref, seg_ref, out_ref):
        out_ref[...] = pe_ref[...][:, None, :] + seg_ref[...][None, :, :]

    comb3 = pl.pallas_call(
        body,
        out_shape=jax.ShapeDtypeStruct((Ls, S, D), jnp.float32),
    )(pe, seg_table)
    return comb3.reshape(Ls * S, D)


def _sc_embed(seq_flat, lab_flat, token_table, comb, seq_len):
    R = seq_flat.shape[0]
    D = token_table.shape[1]
    CH = seq_len
    CPW = R // (_NW * CH)
    NPAD = ((CH + _LANES - 1) // _LANES) * _LANES
    NVEC = NPAD // _LANES
    PIECES = []
    o = 0
    while o < CH:
        n = min(128, CH - o)
        PIECES.append((o, n))
        o += n

    mesh = plsc.VectorSubcoreMesh(core_axis_name="c", subcore_axis_name="s")

    @functools.partial(
        pl.kernel,
        out_type=jax.ShapeDtypeStruct((R, D), jnp.float32),
        mesh=mesh,
        scratch_types=[
            pltpu.VMEM((2, NPAD), jnp.int32),
            pltpu.VMEM((2, NPAD), jnp.int32),
            pltpu.VMEM((2, NPAD), jnp.int32),
            pltpu.VMEM((NPAD,), jnp.int32),
            pltpu.VMEM((2, CH, D), jnp.float32),
            pltpu.VMEM((2, CH, D), jnp.float32),
            pltpu.SemaphoreType.DMA,
            pltpu.SemaphoreType.DMA,
        ],
    )
    def k(seq_hbm, lab_hbm, tok_hbm, comb_hbm, out_hbm,
          tok_idx, lab_v, cidx, lpat, tokbuf, cmbbuf, sem0, sem1):
        wid = lax.axis_index("s") * _NC + lax.axis_index("c")
        row0 = wid * (CPW * CH)
        sems = (sem0, sem1)

        for j in range(NVEC):
            lpat[pl.ds(j * _LANES, _LANES)] = (
                lax.iota(jnp.int32, _LANES) + (j * _LANES)) * 64

        def gather_descs(b):
            ds = []
            for (o, n) in PIECES:
                ds.append((tok_hbm.at[tok_idx.at[b, pl.ds(o, n)]],
                           tokbuf.at[b, pl.ds(o, n)]))
                ds.append((comb_hbm.at[cidx.at[b, pl.ds(o, n)]],
                           cmbbuf.at[b, pl.ds(o, n)]))
            return ds

        def fetch_and_fire(g, b):
            off = row0 + g * CH
            pltpu.sync_copy(seq_hbm.at[pl.ds(off, CH)],
                            tok_idx.at[b, pl.ds(0, CH)])
            pltpu.sync_copy(lab_hbm.at[pl.ds(off, CH)],
                            lab_v.at[b, pl.ds(0, CH)])
            for j in range(NVEC):
                s = pl.ds(j * _LANES, _LANES)
                cidx[b, s] = lab_v[b, s] + lpat[s]
            for (src, dst) in gather_descs(b):
                pltpu.async_copy(src, dst, sems[b])

        def compute_store(g, b):
            for (src, dst) in gather_descs(b):
                pltpu.make_async_copy(src, dst, sems[b]).wait()

            def addrow(r, carry):
                for c in range(D // _LANES):
                    s = pl.ds(c * _LANES, _LANES)
                    tokbuf[b, r, s] = tokbuf[b, r, s] + cmbbuf[b, r, s]
                return carry
            lax.fori_loop(0, CH, addrow, 0)
            off = row0 + g * CH
            pltpu.sync_copy(tokbuf.at[b], out_hbm.at[pl.ds(off, CH)])

        fetch_and_fire(0, 0)

        def pair(p, carry):
            g0 = p * 2
            fetch_and_fire(g0 + 1, 1)
            compute_store(g0, 0)

            @pl.when(g0 + 2 < CPW)
            def _():
                fetch_and_fire(g0 + 2, 0)

            compute_store(g0 + 1, 1)
            return carry

        lax.fori_loop(0, CPW // 2, pair, 0)

    return k(seq_flat, lab_flat, token_table, comb)


def kernel(sequence, segment_label, token_table, seg_table):
    B, L = sequence.shape
    D = token_table.shape[1]

    position = jnp.arange(L, dtype=jnp.float32)[:, None]
    div_term = jnp.exp(jnp.arange(0, D, 2, dtype=jnp.float32)
                       * -(math.log(10000.0) / D))
    pe = jnp.zeros((L, D), dtype=jnp.float32)
    pe = pe.at[:, 0::2].set(jnp.sin(position * div_term))
    pe = pe.at[:, 1::2].set(jnp.cos(position * div_term))

    comb = _make_comb(pe, seg_table)
    out = _sc_embed(sequence.reshape(-1).astype(jnp.int32),
                    segment_label.reshape(-1).astype(jnp.int32),
                    token_table, comb, L)
    return out.reshape(B, L, D)


# speedup vs baseline: 2.4757x; 2.4757x over previous
import functools
import math

import jax
import jax.numpy as jnp
from jax import lax
from jax.experimental import pallas as pl
from jax.experimental.pallas import tpu as pltpu
from jax.experimental.pallas import tpu_sc as plsc

_NC = 2
_NS = 16
_NW = _NC * _NS
_LANES = 16


def _make_comb(pe, seg_table):
    Ls, D = pe.shape
    S = seg_table.shape[0]

    def body(pe_ref, seg_ref, out_ref):
        out_ref[...] = pe_ref[...][:, None, :] + seg_ref[...][None, :, :]

    comb3 = pl.pallas_call(
        body,
        out_shape=jax.ShapeDtypeStruct((Ls, S, D), jnp.float32),
    )(pe, seg_table)
    return comb3.reshape(Ls * S, D)


def _sc_embed(seq_flat, lab_flat, token_table, comb, seq_len):
    R = seq_flat.shape[0]
    D = token_table.shape[1]
    CH = seq_len
    CPW = R // (_NW * CH)
    NPAD = ((CH + _LANES - 1) // _LANES) * _LANES
    NVEC = NPAD // _LANES
    PIECES = []
    o = 0
    while o < CH:
        n = min(128, CH - o)
        PIECES.append((o, n))
        o += n

    mesh = plsc.VectorSubcoreMesh(core_axis_name="c", subcore_axis_name="s")

    @functools.partial(
        pl.kernel,
        out_type=jax.ShapeDtypeStruct((R, D), jnp.float32),
        mesh=mesh,
        compiler_params=pltpu.CompilerParams(use_tc_tiling_on_sc=False),
        scratch_types=[
            pltpu.VMEM((NPAD,), jnp.int32),
            pltpu.VMEM((NPAD,), jnp.int32),
            pltpu.VMEM((NPAD,), jnp.int32),
            pltpu.VMEM((NPAD,), jnp.int32),
            pltpu.VMEM((NPAD,), jnp.int32),
            pltpu.VMEM((NPAD,), jnp.int32),
            pltpu.VMEM((NPAD,), jnp.int32),
            pltpu.VMEM((CH, D), jnp.float32),
            pltpu.VMEM((CH, D), jnp.float32),
            pltpu.VMEM((CH, D), jnp.float32),
            pltpu.VMEM((CH, D), jnp.float32),
            pltpu.SemaphoreType.DMA,
            pltpu.SemaphoreType.DMA,
        ],
    )
    def k(seq_hbm, lab_hbm, tok_hbm, comb_hbm, out_hbm,
          tok_idx0, tok_idx1, lab_v0, lab_v1, cidx0, cidx1, lpat,
          tokbuf0, tokbuf1, cmbbuf0, cmbbuf1, sem0, sem1):
        wid = lax.axis_index("s") * _NC + lax.axis_index("c")
        row0 = wid * (CPW * CH)
        sems = (sem0, sem1)
        tok_idx = (tok_idx0, tok_idx1)
        lab_v = (lab_v0, lab_v1)
        cidx = (cidx0, cidx1)
        tokbuf = (tokbuf0, tokbuf1)
        cmbbuf = (cmbbuf0, cmbbuf1)

        for j in range(NVEC):
            lpat[pl.ds(j * _LANES, _LANES)] = (
                lax.iota(jnp.int32, _LANES) + (j * _LANES)) * 64

        def gather_descs(b):
            ds = []
            for (o, n) in PIECES:
                ds.append((tok_hbm.at[tok_idx[b].at[pl.ds(o, n)]],
                           tokbuf[b].at[pl.ds(o, n)]))
                ds.append((comb_hbm.at[cidx[b].at[pl.ds(o, n)]],
                           cmbbuf[b].at[pl.ds(o, n)]))
            return ds

        def fetch_and_fire(g, b):
            off = row0 + g * CH
            pltpu.sync_copy(seq_hbm.at[pl.ds(off, CH)],
                            tok_idx[b].at[pl.ds(0, CH)])
            pltpu.sync_copy(lab_hbm.at[pl.ds(off, CH)],
                            lab_v[b].at[pl.ds(0, CH)])
            for j in range(NVEC):
                s = pl.ds(j * _LANES, _LANES)
                cidx[b][s] = lab_v[b][s] + lpat[s]
            for (src, dst) in gather_descs(b):
                pltpu.async_copy(src, dst, sems[b])

        def compute_store(g, b):
            for (src, dst) in gather_descs(b):
                pltpu.make_async_copy(src, dst, sems[b]).wait()

            def addrow(r, carry):
                for c in range(D // _LANES):
                    s = pl.ds(c * _LANES, _LANES)
                    tokbuf[b][r, s] = tokbuf[b][r, s] + cmbbuf[b][r, s]
                return carry
            lax.fori_loop(0, CH, addrow, 0)
            off = row0 + g * CH
            pltpu.sync_copy(tokbuf[b], out_hbm.at[pl.ds(off, CH)])

        fetch_and_fire(0, 0)

        def pair(p, carry):
            g0 = p * 2
            fetch_and_fire(g0 + 1, 1)
            compute_store(g0, 0)

            @pl.when(g0 + 2 < CPW)
            def _():
                fetch_and_fire(g0 + 2, 0)

            compute_store(g0 + 1, 1)
            return carry

        lax.fori_loop(0, CPW // 2, pair, 0)

    return k(seq_flat, lab_flat, token_table, comb)


def kernel(sequence, segment_label, token_table, seg_table):
    B, L = sequence.shape
    D = token_table.shape[1]

    position = jnp.arange(L, dtype=jnp.float32)[:, None]
    div_term = jnp.exp(jnp.arange(0, D, 2, dtype=jnp.float32)
                       * -(math.log(10000.0) / D))
    pe = jnp.zeros((L, D), dtype=jnp.float32)
    pe = pe.at[:, 0::2].set(jnp.sin(position * div_term))
    pe = pe.at[:, 1::2].set(jnp.cos(position * div_term))

    comb = _make_comb(pe, seg_table)
    out = _sc_embed(sequence.reshape(-1).astype(jnp.int32),
                    segment_label.reshape(-1).astype(jnp.int32),
                    token_table, comb, L)
    return out.reshape(B, L, D)


# speedup vs baseline: 2.9824x; 1.2047x over previous
import functools
import math

import jax
import jax.numpy as jnp
from jax import lax
from jax.experimental import pallas as pl
from jax.experimental.pallas import tpu as pltpu
from jax.experimental.pallas import tpu_sc as plsc

_NC = 2
_NS = 16
_NW = _NC * _NS
_LANES = 16


def _transpose_table(token_table):
    V, D = token_table.shape
    tableT = token_table.T
    BT = 2048
    H1 = (V // 2) // BT * BT
    NB = pl.cdiv(V - H1, BT)
    HO = V - H1

    def body(top_ref, bot_ref, out_ref):
        out_ref[...] = jnp.concatenate([top_ref[...].T, bot_ref[...].T],
                                       axis=1)

    packed = pl.pallas_call(
        body,
        grid=(NB,),
        in_specs=[pl.BlockSpec((D, BT), lambda i: (0, i)),
                  pl.BlockSpec((D, BT), lambda i: (0, i + H1 // BT))],
        out_specs=pl.BlockSpec((BT, 2 * D), lambda i: (i, 0)),
        out_shape=jax.ShapeDtypeStruct((HO, 2 * D), jnp.float32),
    )(tableT, tableT)
    return packed.reshape(2 * HO, D), H1


def _make_comb(pe, seg_table):
    Ls, D = pe.shape
    S = seg_table.shape[0]

    def body(pe_ref, seg_ref, out_ref):
        out_ref[...] = pe_ref[...][:, None, :] + seg_ref[...][None, :, :]

    comb3 = pl.pallas_call(
        body,
        out_shape=jax.ShapeDtypeStruct((Ls, S, D), jnp.float32),
    )(pe, seg_table)
    return comb3.reshape(Ls * S, D)


def _sc_embed(seq_flat, lab_flat, token_table, comb, seq_len, split):
    R = seq_flat.shape[0]
    D = token_table.shape[1]
    CH = seq_len
    CPW = R // (_NW * CH)
    NPAD = ((CH + _LANES - 1) // _LANES) * _LANES
    NVEC = NPAD // _LANES
    PIECES = []
    o = 0
    while o < CH:
        n = min(128, CH - o)
        PIECES.append((o, n))
        o += n

    mesh = plsc.VectorSubcoreMesh(core_axis_name="c", subcore_axis_name="s")

    @functools.partial(
        pl.kernel,
        out_type=jax.ShapeDtypeStruct((R // CH, CH, D), jnp.float32),
        mesh=mesh,
        compiler_params=pltpu.CompilerParams(use_tc_tiling_on_sc=False),
        scratch_types=[
            pltpu.VMEM((NPAD,), jnp.int32),
            pltpu.VMEM((NPAD,), jnp.int32),
            pltpu.VMEM((NPAD,), jnp.int32),
            pltpu.VMEM((NPAD,), jnp.int32),
            pltpu.VMEM((NPAD,), jnp.int32),
            pltpu.VMEM((NPAD,), jnp.int32),
            pltpu.VMEM((NPAD,), jnp.int32),
            pltpu.VMEM((CH, D), jnp.float32),
            pltpu.VMEM((CH, D), jnp.float32),
            pltpu.VMEM((CH, D), jnp.float32),
            pltpu.VMEM((CH, D), jnp.float32),
            pltpu.SemaphoreType.DMA,
            pltpu.SemaphoreType.DMA,
        ],
    )
    def k(seq_hbm, lab_hbm, tok_hbm, comb_hbm, out_hbm,
          tok_idx0, tok_idx1, lab_v0, lab_v1, cidx0, cidx1, lpat,
          tokbuf0, tokbuf1, cmbbuf0, cmbbuf1, sem0, sem1):
        wid = lax.axis_index("s") * _NC + lax.axis_index("c")
        row0 = wid * (CPW * CH)
        sems = (sem0, sem1)
        tok_idx = (tok_idx0, tok_idx1)
        lab_v = (lab_v0, lab_v1)
        cidx = (cidx0, cidx1)
        tokbuf = (tokbuf0, tokbuf1)
        cmbbuf = (cmbbuf0, cmbbuf1)

        for j in range(NVEC):
            lpat[pl.ds(j * _LANES, _LANES)] = (
                lax.iota(jnp.int32, _LANES) + (j * _LANES)) * 64

        def gather_descs(b):
            ds = []
            for (o, n) in PIECES:
                ds.append((tok_hbm.at[tok_idx[b].at[pl.ds(o, n)]],
                           tokbuf[b].at[pl.ds(o, n)]))
                ds.append((comb_hbm.at[cidx[b].at[pl.ds(o, n)]],
                           cmbbuf[b].at[pl.ds(o, n)]))
            return ds

        VH = split

        def fetch_and_fire(g, b):
            off = row0 + g * CH
            pltpu.sync_copy(seq_hbm.at[pl.ds(off, CH)],
                            tok_idx[b].at[pl.ds(0, CH)])
            pltpu.sync_copy(lab_hbm.at[pl.ds(off, CH)],
                            lab_v[b].at[pl.ds(0, CH)])
            for j in range(NVEC):
                s = pl.ds(j * _LANES, _LANES)
                cidx[b][s] = lab_v[b][s] + lpat[s]
                t = tok_idx[b][s]
                tok_idx[b][s] = jnp.where(t >= VH, t * 2 - (2 * VH - 1), t * 2)
            for (src, dst) in gather_descs(b):
                pltpu.async_copy(src, dst, sems[b])

        def compute_store(g, b):
            for (src, dst) in gather_descs(b):
                pltpu.make_async_copy(src, dst, sems[b]).wait()

            def addrow(r, carry):
                for c in range(D // _LANES):
                    s = pl.ds(c * _LANES, _LANES)
                    tokbuf[b][r, s] = tokbuf[b][r, s] + cmbbuf[b][r, s]
                return carry
            lax.fori_loop(0, CH, addrow, 0)
            pltpu.sync_copy(tokbuf[b], out_hbm.at[row0 // CH + g])

        fetch_and_fire(0, 0)

        def pair(p, carry):
            g0 = p * 2
            fetch_and_fire(g0 + 1, 1)
            compute_store(g0, 0)

            @pl.when(g0 + 2 < CPW)
            def _():
                fetch_and_fire(g0 + 2, 0)

            compute_store(g0 + 1, 1)
            return carry

        lax.fori_loop(0, CPW // 2, pair, 0)

    return k(seq_flat, lab_flat, token_table, comb)


def kernel(sequence, segment_label, token_table, seg_table):
    B, L = sequence.shape
    D = token_table.shape[1]

    position = jnp.arange(L, dtype=jnp.float32)[:, None]
    div_term = jnp.exp(jnp.arange(0, D, 2, dtype=jnp.float32)
                       * -(math.log(10000.0) / D))
    pe = jnp.zeros((L, D), dtype=jnp.float32)
    pe = pe.at[:, 0::2].set(jnp.sin(position * div_term))
    pe = pe.at[:, 1::2].set(jnp.cos(position * div_term))

    comb = _make_comb(pe, seg_table)
    table_lin, split = _transpose_table(token_table)
    out = _sc_embed(sequence.reshape(-1).astype(jnp.int32),
                    segment_label.reshape(-1).astype(jnp.int32),
                    table_lin, comb, L, split)
    return out
